# baseline (speedup 1.0000x reference)
import functools

import jax
import jax.numpy as jnp
from jax import lax
from jax.experimental import pallas as pl
from jax.experimental.pallas import tpu as pltpu
import jax.experimental.pallas.tpu_sc as plsc

N = 50000
E = 800000
T = 4
IN = 8
OUT = 8
B = 4
D = B * OUT
NC = 2
NS = 16
NW = NC * NS
KB = 128
NBLK = 196
CBLK = 28
NCHUNK = NBLK // CBLK
EPT = NBLK * KB
EP = NW * EPT
NROWSP = 50048
RPT = NROWSP // NS
ZR = 136
NBUF = 4
NR128 = NROWSP * OUT // 128
NB_TC = 1000
GRID = N // NB_TC
K1 = 40


def _xw_body(x_ref, w_ref, o_ref):
    o_ref[...] = jnp.dot(x_ref[...], w_ref[...],
                         preferred_element_type=jnp.float32)


def _xw_call(xt40, wmat40):
    return pl.pallas_call(
        _xw_body,
        grid=(GRID,),
        in_specs=[pl.BlockSpec((NB_TC, K1), lambda i: (i, 0)),
                  pl.BlockSpec((K1, T * D), lambda i: (0, 0))],
        out_specs=pl.BlockSpec((NB_TC, T * D), lambda i: (i, 0)),
        out_shape=jax.ShapeDtypeStruct((N, T * D), jnp.float32),
    )(xt40, wmat40)


_mesh = plsc.VectorSubcoreMesh(core_axis_name="c", subcore_axis_name="s")


@functools.partial(
    pl.kernel,
    out_type=jax.ShapeDtypeStruct((NC, B, NROWSP, OUT), jnp.float32),
    mesh=_mesh,
    scratch_types=[
        pltpu.VMEM((CBLK, KB), jnp.int32),
        pltpu.VMEM((CBLK, KB), jnp.int32),
        [pltpu.VMEM((KB, D), jnp.float32) for _ in range(NBUF)],
        pltpu.VMEM((ZR, D), jnp.float32),
        pltpu.VMEM_SHARED((NROWSP, D), jnp.float32),
        [pltpu.SemaphoreType.DMA for _ in range(NBUF)],
    ],
    compiler_params=pltpu.CompilerParams(use_tc_tiling_on_sc=False),
)
def _sc_accum(xw_hbm, g_hbm, d_hbm, y_hbm,
              g_v, d_v, rows_v, zb_v, acc_sh, sems):
    c = lax.axis_index("c")
    s = lax.axis_index("s")
    wid = c * NS + s
    z16 = jnp.zeros((16,), jnp.float32)

    def _zrow(i, carry):
        zb_v[i, pl.ds(0, 16)] = z16
        zb_v[i, pl.ds(16, 16)] = z16
        return carry

    lax.fori_loop(0, ZR, _zrow, 0)

    base = s * RPT
    for j in range(RPT // ZR):
        pltpu.sync_copy(zb_v, acc_sh.at[pl.ds(base + j * ZR, ZR)])
    plsc.subcore_barrier()

    def _chunk(q, carry):
        pltpu.sync_copy(g_hbm.at[wid, q], g_v)
        pltpu.sync_copy(d_hbm.at[wid, q], d_v)
        for i in range(NBUF):
            pltpu.async_copy(xw_hbm.at[g_v.at[i]], rows_v[i], sems[i])

        def _group(p, inner):
            for i in range(NBUF):
                j = NBUF * p + i
                pltpu.make_async_copy(
                    xw_hbm.at[g_v.at[j]], rows_v[i], sems[i]).wait()
                pltpu.sync_copy(rows_v[i], acc_sh.at[d_v.at[j]], add=True)

                @pl.when(j + NBUF < CBLK)
                def _():
                    pltpu.async_copy(
                        xw_hbm.at[g_v.at[j + NBUF]], rows_v[i], sems[i])
            return inner

        lax.fori_loop(0, CBLK // NBUF, _group, 0)
        return carry

    lax.fori_loop(0, NCHUNK, _chunk, 0)
    plsc.subcore_barrier()
    for b in range(B):
        pltpu.sync_copy(acc_sh.at[pl.ds(base, RPT), pl.ds(b * OUT, OUT)],
                        y_hbm.at[c, b, pl.ds(base, RPT)])


def _fin_body(y_ref, o_ref):
    o_ref[0] = y_ref[0, 0] + y_ref[1, 0]


def _fin_call(ypart128):
    return pl.pallas_call(
        _fin_body,
        grid=(B,),
        in_specs=[pl.BlockSpec((NC, 1, NR128, 128), lambda i: (0, i, 0, 0))],
        out_specs=pl.BlockSpec((1, NR128, 128), lambda i: (i, 0, 0)),
        out_shape=jax.ShapeDtypeStruct((B, NR128, 128), jnp.float32),
    )(ypart128)


def kernel(input, edge_index, edge_types, weight_vector, bias_vector):
    x = input.astype(jnp.float32)
    W = weight_vector.reshape(T, OUT, IN)
    bias = bias_vector.reshape(T, OUT)

    Wt = jnp.transpose(W, (2, 0, 1))
    eye = jnp.eye(B, dtype=jnp.float32)
    wfull = eye[:, None, None, :, None] * Wt[None, :, :, None, :]
    wmat = wfull.reshape(B * IN, T * B * OUT)
    brow = jnp.broadcast_to(bias[:, None, :], (T, B, OUT)).reshape(1, T * B * OUT)
    wmat40 = jnp.concatenate(
        [wmat, brow,
         jnp.zeros((K1 - B * IN - 1, T * B * OUT), jnp.float32)], axis=0)

    xt = jnp.transpose(x, (1, 0, 2)).reshape(N, B * IN)
    xt40 = jnp.concatenate(
        [xt, jnp.ones((N, 1), jnp.float32),
         jnp.zeros((N, K1 - B * IN - 1), jnp.float32)], axis=1)

    xw = _xw_call(xt40, wmat40).reshape(N * T, D)

    src = edge_index[1].astype(jnp.int32)
    dst = edge_index[0].astype(jnp.int32)
    typ = edge_types.astype(jnp.int32)
    g = src * T + typ
    gp = jnp.concatenate(
        [g, jnp.zeros((EP - E,), jnp.int32)]).reshape(NW, NCHUNK, CBLK, KB)
    dp = jnp.concatenate(
        [dst, jnp.full((EP - E,), N, jnp.int32)]).reshape(NW, NCHUNK, CBLK, KB)

    ypart = _sc_accum(xw, gp, dp)
    ysum = _fin_call(ypart.reshape(NC, B, NR128, 128))
    return ysum.reshape(B, NROWSP, OUT)[:, :N, :]


# speedup vs baseline: 1.3337x; 1.3337x over previous
import functools

import jax
import jax.numpy as jnp
from jax import lax
from jax.experimental import pallas as pl
from jax.experimental.pallas import tpu as pltpu
import jax.experimental.pallas.tpu_sc as plsc

N = 50000
E = 800000
T = 4
IN = 8
OUT = 8
B = 4
D = B * OUT
NC = 2
NS = 16
NW = NC * NS
KB = 128
NBLKG = E // KB
CBLK = 25
NCH = NBLKG // CBLK
CHF = NCH // NW
CHX = NCH - NW * CHF
NBUF = 5
RPT = N // NS
ZR = 125
NPW = 1562
NREM = N - NW * NPW
NB_TC = 1000
GRID = N // NB_TC
K1 = 40


def _xw_body(x_ref, w_ref, o_ref):
    o_ref[...] = jnp.dot(x_ref[...], w_ref[...],
                         preferred_element_type=jnp.float32)


def _xw_call(xt40, wmat40):
    return pl.pallas_call(
        _xw_body,
        grid=(GRID,),
        in_specs=[pl.BlockSpec((NB_TC, K1), lambda i: (i, 0)),
                  pl.BlockSpec((K1, T * D), lambda i: (0, 0))],
        out_specs=pl.BlockSpec((NB_TC, T * D), lambda i: (i, 0)),
        out_shape=jax.ShapeDtypeStruct((N, T * D), jnp.float32),
    )(xt40, wmat40)


_mesh = plsc.VectorSubcoreMesh(core_axis_name="c", subcore_axis_name="s")


@functools.partial(
    pl.kernel,
    out_type=jax.ShapeDtypeStruct((NC, N, D), jnp.float32),
    mesh=_mesh,
    scratch_types=[
        pltpu.VMEM((CBLK, KB), jnp.int32),
        pltpu.VMEM((CBLK, KB), jnp.int32),
        [pltpu.VMEM((KB, D), jnp.float32) for _ in range(NBUF)],
        pltpu.VMEM((ZR, D), jnp.float32),
        pltpu.VMEM_SHARED((N, D), jnp.float32),
        [pltpu.SemaphoreType.DMA for _ in range(NBUF)],
    ],
    compiler_params=pltpu.CompilerParams(use_tc_tiling_on_sc=False),
)
def _sc_accum(xw_hbm, g_hbm, d_hbm, y_hbm,
              g_v, d_v, rows_v, zb_v, acc_sh, sems):
    c = lax.axis_index("c")
    s = lax.axis_index("s")
    wid = c * NS + s
    z16 = jnp.zeros((16,), jnp.float32)

    def _zrow(i, carry):
        zb_v[i, pl.ds(0, 16)] = z16
        zb_v[i, pl.ds(16, 16)] = z16
        return carry

    lax.fori_loop(0, ZR, _zrow, 0)

    base = s * RPT
    for j in range(RPT // ZR):
        pltpu.sync_copy(zb_v, acc_sh.at[pl.ds(base + j * ZR, ZR)])
    plsc.subcore_barrier()

    ch0 = wid * CHF + jnp.minimum(wid, CHX)
    nch = jnp.where(wid < CHX, CHF + 1, CHF)

    def _chunk(q, carry):
        blk0 = (ch0 + q) * CBLK
        pltpu.sync_copy(g_hbm.at[pl.ds(blk0, CBLK)], g_v)
        pltpu.sync_copy(d_hbm.at[pl.ds(blk0, CBLK)], d_v)
        for i in range(NBUF):
            pltpu.async_copy(xw_hbm.at[g_v.at[i]], rows_v[i], sems[i])

        def _group(p, inner):
            for i in range(NBUF):
                j = NBUF * p + i
                pltpu.make_async_copy(
                    xw_hbm.at[g_v.at[j]], rows_v[i], sems[i]).wait()
                pltpu.sync_copy(rows_v[i], acc_sh.at[d_v.at[j]], add=True)

                @pl.when(j + NBUF < CBLK)
                def _():
                    pltpu.async_copy(
                        xw_hbm.at[g_v.at[j + NBUF]], rows_v[i], sems[i])
            return inner

        lax.fori_loop(0, CBLK // NBUF, _group, 0)
        return carry

    lax.fori_loop(0, nch, _chunk, 0)
    plsc.subcore_barrier()
    pltpu.sync_copy(acc_sh.at[pl.ds(base, RPT)],
                    y_hbm.at[c, pl.ds(base, RPT)])


@functools.partial(
    pl.kernel,
    out_type=jax.ShapeDtypeStruct((B, N, OUT), jnp.float32),
    mesh=_mesh,
    scratch_types=[
        pltpu.VMEM((NPW, D), jnp.float32),
        pltpu.VMEM((NPW, D), jnp.float32),
    ],
    compiler_params=pltpu.CompilerParams(use_tc_tiling_on_sc=False),
)
def _sc_merge(yp_hbm, y_hbm, p0_v, p1_v):
    c = lax.axis_index("c")
    s = lax.axis_index("s")
    wid = c * NS + s
    nbase = wid * NPW
    pltpu.sync_copy(yp_hbm.at[0, pl.ds(nbase, NPW)], p0_v)
    pltpu.sync_copy(yp_hbm.at[1, pl.ds(nbase, NPW)], p1_v)

    def _addrow(r, carry):
        p0_v[r, pl.ds(0, 16)] = p0_v[r, pl.ds(0, 16)] + p1_v[r, pl.ds(0, 16)]
        p0_v[r, pl.ds(16, 16)] = p0_v[r, pl.ds(16, 16)] + p1_v[r, pl.ds(16, 16)]
        return carry

    lax.fori_loop(0, NPW, _addrow, 0)
    for b in range(B):
        pltpu.sync_copy(p0_v.at[pl.ds(0, NPW), pl.ds(b * OUT, OUT)],
                        y_hbm.at[b, pl.ds(nbase, NPW)])

    @pl.when(wid == 0)
    def _():
        rbase = NW * NPW
        pltpu.sync_copy(yp_hbm.at[0, pl.ds(rbase, NREM)],
                        p0_v.at[pl.ds(0, NREM)])
        pltpu.sync_copy(yp_hbm.at[1, pl.ds(rbase, NREM)],
                        p1_v.at[pl.ds(0, NREM)])

        def _addrem(r, carry):
            p0_v[r, pl.ds(0, 16)] = (p0_v[r, pl.ds(0, 16)]
                                     + p1_v[r, pl.ds(0, 16)])
            p0_v[r, pl.ds(16, 16)] = (p0_v[r, pl.ds(16, 16)]
                                      + p1_v[r, pl.ds(16, 16)])
            return carry

        lax.fori_loop(0, NREM, _addrem, 0)
        for b in range(B):
            pltpu.sync_copy(p0_v.at[pl.ds(0, NREM), pl.ds(b * OUT, OUT)],
                            y_hbm.at[b, pl.ds(rbase, NREM)])


def kernel(input, edge_index, edge_types, weight_vector, bias_vector):
    x = input.astype(jnp.float32)
    W = weight_vector.reshape(T, OUT, IN)
    bias = bias_vector.reshape(T, OUT)

    Wt = jnp.transpose(W, (2, 0, 1))
    eye = jnp.eye(B, dtype=jnp.float32)
    wfull = eye[:, None, None, :, None] * Wt[None, :, :, None, :]
    wmat = wfull.reshape(B * IN, T * B * OUT)
    brow = jnp.broadcast_to(bias[:, None, :], (T, B, OUT)).reshape(1, T * B * OUT)
    wmat40 = jnp.concatenate(
        [wmat, brow,
         jnp.zeros((K1 - B * IN - 1, T * B * OUT), jnp.float32)], axis=0)

    xt = jnp.transpose(x, (1, 0, 2)).reshape(N, B * IN)
    xt40 = jnp.concatenate(
        [xt, jnp.ones((N, 1), jnp.float32),
         jnp.zeros((N, K1 - B * IN - 1), jnp.float32)], axis=1)

    xw = _xw_call(xt40, wmat40).reshape(N * T, D)

    src = edge_index[1].astype(jnp.int32)
    dst = edge_index[0].astype(jnp.int32)
    typ = edge_types.astype(jnp.int32)
    gp = (src * T + typ).reshape(NBLKG, KB)
    dp = dst.reshape(NBLKG, KB)

    ypart = _sc_accum(xw, gp, dp)
    return _sc_merge(ypart)


# speedup vs baseline: 1.8899x; 1.4170x over previous
import functools

import jax
import jax.numpy as jnp
from jax import lax
from jax.experimental import pallas as pl
from jax.experimental.pallas import tpu as pltpu
import jax.experimental.pallas.tpu_sc as plsc

N = 50000
E = 800000
T = 4
IN = 8
OUT = 8
B = 4
D = B * OUT
NC = 2
NS = 16
NW = NC * NS
KB = 128
NBLKG = E // KB
CBLK = 25
NCH = NBLKG // CBLK
CHF = NCH // NW
CHX = NCH - NW * CHF
NBUF = 5
RPT = N // NS
ZR = 125
OCT = (N // 8) // NW
OCTX = (N // 8) - NW * OCT
MC = 784
NB_TC = 1000
GRID = N // NB_TC
K1 = 40


def _xw_body(x_ref, w_ref, o_ref):
    o_ref[...] = jnp.dot(x_ref[...], w_ref[...],
                         preferred_element_type=jnp.float32)


def _xw_call(xt40, wmat40):
    return pl.pallas_call(
        _xw_body,
        grid=(GRID,),
        in_specs=[pl.BlockSpec((NB_TC, K1), lambda i: (i, 0)),
                  pl.BlockSpec((K1, T * D), lambda i: (0, 0))],
        out_specs=pl.BlockSpec((NB_TC, T * D), lambda i: (i, 0)),
        out_shape=jax.ShapeDtypeStruct((N, T * D), jnp.float32),
    )(xt40, wmat40)


_mesh = plsc.VectorSubcoreMesh(core_axis_name="c", subcore_axis_name="s")


@functools.partial(
    pl.kernel,
    out_type=jax.ShapeDtypeStruct((NC, N, D), jnp.float32),
    mesh=_mesh,
    scratch_types=[
        pltpu.VMEM((CBLK, KB), jnp.int32),
        pltpu.VMEM((CBLK, KB), jnp.int32),
        [pltpu.VMEM((KB, D), jnp.float32) for _ in range(NBUF)],
        pltpu.VMEM((ZR, D), jnp.float32),
        pltpu.VMEM_SHARED((N, D), jnp.float32),
        [pltpu.SemaphoreType.DMA for _ in range(NBUF)],
    ],
    compiler_params=pltpu.CompilerParams(use_tc_tiling_on_sc=False),
)
def _sc_accum(xw_hbm, g_hbm, d_hbm, y_hbm,
              g_v, d_v, rows_v, zb_v, acc_sh, sems):
    c = lax.axis_index("c")
    s = lax.axis_index("s")
    wid = c * NS + s
    z16 = jnp.zeros((16,), jnp.float32)

    def _zrow(i, carry):
        zb_v[i, pl.ds(0, 16)] = z16
        zb_v[i, pl.ds(16, 16)] = z16
        return carry

    lax.fori_loop(0, ZR, _zrow, 0)

    base = s * RPT
    for j in range(RPT // ZR):
        pltpu.sync_copy(zb_v, acc_sh.at[pl.ds(base + j * ZR, ZR)])
    plsc.subcore_barrier()

    ch0 = wid * CHF + jnp.minimum(wid, CHX)
    nch = jnp.where(wid < CHX, CHF + 1, CHF)

    def _chunk(q, carry):
        blk0 = (ch0 + q) * CBLK
        pltpu.sync_copy(g_hbm.at[pl.ds(blk0, CBLK)], g_v)
        pltpu.sync_copy(d_hbm.at[pl.ds(blk0, CBLK)], d_v)
        for i in range(NBUF):
            pltpu.async_copy(xw_hbm.at[g_v.at[i]], rows_v[i], sems[i])

        def _group(p, inner):
            for i in range(NBUF):
                j = NBUF * p + i
                pltpu.make_async_copy(
                    xw_hbm.at[g_v.at[j]], rows_v[i], sems[i]).wait()
                pltpu.sync_copy(rows_v[i], acc_sh.at[d_v.at[j]], add=True)

                @pl.when(j + NBUF < CBLK)
                def _():
                    pltpu.async_copy(
                        xw_hbm.at[g_v.at[j + NBUF]], rows_v[i], sems[i])
            return inner

        lax.fori_loop(0, CBLK // NBUF, _group, 0)
        return carry

    lax.fori_loop(0, nch, _chunk, 0)
    plsc.subcore_barrier()
    pltpu.sync_copy(acc_sh.at[pl.ds(base, RPT)],
                    y_hbm.at[c, pl.ds(base, RPT)])


def _merge_chunk(yp_hbm, y_hbm, p0_v, p1_v, pt_v, nb, rows):
    pltpu.sync_copy(yp_hbm.at[0, pl.ds(nb, rows)], p0_v.at[pl.ds(0, rows)])
    pltpu.sync_copy(yp_hbm.at[1, pl.ds(nb, rows)], p1_v.at[pl.ds(0, rows)])
    lo = lax.iota(jnp.int32, 16)
    hi = lo + 16

    def _row(r, carry):
        v0 = p0_v[r, pl.ds(0, 16)] + p1_v[r, pl.ds(0, 16)]
        v1 = p0_v[r, pl.ds(16, 16)] + p1_v[r, pl.ds(16, 16)]
        rr = jnp.full((16,), 0, jnp.int32) + r
        plsc.store_scatter(pt_v, [lo, rr], v0)
        plsc.store_scatter(pt_v, [hi, rr], v1)
        return carry

    lax.fori_loop(0, rows, _row, 0)
    for b in range(B):
        pltpu.sync_copy(pt_v.at[pl.ds(b * OUT, OUT), pl.ds(0, rows)],
                        y_hbm.at[b, pl.ds(0, OUT), pl.ds(nb, rows)])


@functools.partial(
    pl.kernel,
    out_type=jax.ShapeDtypeStruct((B, OUT, N), jnp.float32),
    mesh=_mesh,
    scratch_types=[
        pltpu.VMEM((MC, D), jnp.float32),
        pltpu.VMEM((MC, D), jnp.float32),
        pltpu.VMEM((D, MC), jnp.float32),
    ],
    compiler_params=pltpu.CompilerParams(use_tc_tiling_on_sc=False,
                                         needs_layout_passes=False),
)
def _sc_merge(yp_hbm, y_hbm, p0_v, p1_v, pt_v):
    c = lax.axis_index("c")
    s = lax.axis_index("s")
    wid = c * NS + s

    @pl.when(wid < OCTX)
    def _():
        nb = wid * (OCT + 1) * 8
        _merge_chunk(yp_hbm, y_hbm, p0_v, p1_v, pt_v, nb, MC)
        _merge_chunk(yp_hbm, y_hbm, p0_v, p1_v, pt_v, nb + MC, MC)

    @pl.when(wid >= OCTX)
    def _():
        nb = (wid * OCT + OCTX) * 8
        _merge_chunk(yp_hbm, y_hbm, p0_v, p1_v, pt_v, nb, MC)
        _merge_chunk(yp_hbm, y_hbm, p0_v, p1_v, pt_v, nb + MC, MC - 8)


def kernel(input, edge_index, edge_types, weight_vector, bias_vector):
    x = input.astype(jnp.float32)
    W = weight_vector.reshape(T, OUT, IN)
    bias = bias_vector.reshape(T, OUT)

    Wt = jnp.transpose(W, (2, 0, 1))
    eye = jnp.eye(B, dtype=jnp.float32)
    wfull = eye[:, None, None, :, None] * Wt[None, :, :, None, :]
    wmat = wfull.reshape(B * IN, T * B * OUT)
    brow = jnp.broadcast_to(bias[:, None, :], (T, B, OUT)).reshape(1, T * B * OUT)
    wmat40 = jnp.concatenate(
        [wmat, brow,
         jnp.zeros((K1 - B * IN - 1, T * B * OUT), jnp.float32)], axis=0)

    xt = jnp.transpose(x, (1, 0, 2)).reshape(N, B * IN)
    xt40 = jnp.concatenate(
        [xt, jnp.ones((N, 1), jnp.float32),
         jnp.zeros((N, K1 - B * IN - 1), jnp.float32)], axis=1)

    xw = _xw_call(xt40, wmat40).reshape(N * T, D)

    src = edge_index[1].astype(jnp.int32)
    dst = edge_index[0].astype(jnp.int32)
    typ = edge_types.astype(jnp.int32)
    gp = (src * T + typ).reshape(NBLKG, KB)
    dp = dst.reshape(NBLKG, KB)

    ypart = _sc_accum(xw, gp, dp)
    return jnp.transpose(_sc_merge(ypart), (0, 2, 1))


# speedup vs baseline: 2.4184x; 1.2797x over previous
import functools

import jax
import jax.numpy as jnp
from jax import lax
from jax.experimental import pallas as pl
from jax.experimental.pallas import tpu as pltpu
import jax.experimental.pallas.tpu_sc as plsc

N = 50000
E = 800000
T = 4
IN = 8
OUT = 8
B = 4
D = B * OUT
NC = 2
NS = 16
NW = NC * NS
KB = 128
NBLKG = E // KB
CBLK = 25
NCH = NBLKG // CBLK
CHF = NCH // NW
CHX = NCH - NW * CHF
NBUF = 5
RPT = N // NS
ZR = 25
OCT = (N // 8) // NW
OCTX = (N // 8) - NW * OCT
MC = 784
NP = 50048
NB_TC = 2176
GRID = NP // NB_TC


def _xw_body(x_ref, w_ref, b_ref, o_ref):
    o_ref[...] = lax.dot_general(
        x_ref[...], w_ref[...], (((0,), (0,)), ((), ())),
        preferred_element_type=jnp.float32) + b_ref[...]


def _xw_call(x2, wmat, brow):
    return pl.pallas_call(
        _xw_body,
        grid=(GRID,),
        in_specs=[pl.BlockSpec((B * IN, NB_TC), lambda i: (0, i)),
                  pl.BlockSpec((B * IN, T * D), lambda i: (0, 0)),
                  pl.BlockSpec((1, T * D), lambda i: (0, 0))],
        out_specs=pl.BlockSpec((NB_TC, T * D), lambda i: (i, 0)),
        out_shape=jax.ShapeDtypeStruct((NP, T * D), jnp.float32),
    )(x2, wmat, brow)


_mesh = plsc.VectorSubcoreMesh(core_axis_name="c", subcore_axis_name="s")


@functools.partial(
    pl.kernel,
    out_type=jax.ShapeDtypeStruct((NC, N, D), jnp.float32),
    mesh=_mesh,
    scratch_types=[
        pltpu.VMEM((CBLK, 2, KB), jnp.int32),
        pltpu.VMEM((CBLK, KB), jnp.int32),
        [pltpu.VMEM((KB, D), jnp.float32) for _ in range(NBUF)],
        pltpu.VMEM((ZR, D), jnp.float32),
        pltpu.VMEM_SHARED((N, D), jnp.float32),
        [pltpu.SemaphoreType.DMA for _ in range(NBUF)],
    ],
    compiler_params=pltpu.CompilerParams(use_tc_tiling_on_sc=False),
)
def _sc_accum(xw_hbm, ei_hbm, t_hbm, y_hbm,
              ei_v, t_v, rows_v, zb_v, acc_sh, sems):
    c = lax.axis_index("c")
    s = lax.axis_index("s")
    wid = c * NS + s
    z16 = jnp.zeros((16,), jnp.float32)

    def _zrow(i, carry):
        zb_v[i, pl.ds(0, 16)] = z16
        zb_v[i, pl.ds(16, 16)] = z16
        return carry

    lax.fori_loop(0, ZR, _zrow, 0)

    base = s * RPT

    def _zcp(j, carry):
        pltpu.sync_copy(zb_v, acc_sh.at[pl.ds(base + j * ZR, ZR)])
        return carry

    lax.fori_loop(0, RPT // ZR, _zcp, 0)
    plsc.subcore_barrier()

    ch0 = wid * CHF + jnp.minimum(wid, CHX)
    nch = jnp.where(wid < CHX, CHF + 1, CHF)

    def _chunk(q, carry):
        blk0 = (ch0 + q) * CBLK
        pltpu.sync_copy(ei_hbm.at[pl.ds(blk0, CBLK)], ei_v)
        pltpu.sync_copy(t_hbm.at[pl.ds(blk0, CBLK)], t_v)

        def _g(j, inner):
            for h in range(KB // 16):
                sl = pl.ds(16 * h, 16)
                ei_v[j, 1, sl] = ei_v[j, 1, sl] * T + t_v[j, sl]
            return inner

        lax.fori_loop(0, CBLK, _g, 0)

        for i in range(NBUF):
            pltpu.async_copy(xw_hbm.at[ei_v.at[i, 1]], rows_v[i], sems[i])

        def _group(p, inner):
            for i in range(NBUF):
                j = NBUF * p + i
                pltpu.make_async_copy(
                    xw_hbm.at[ei_v.at[j, 1]], rows_v[i], sems[i]).wait()
                pltpu.sync_copy(rows_v[i], acc_sh.at[ei_v.at[j, 0]], add=True)

                @pl.when(j + NBUF < CBLK)
                def _():
                    pltpu.async_copy(
                        xw_hbm.at[ei_v.at[j + NBUF, 1]], rows_v[i], sems[i])
            return inner

        lax.fori_loop(0, CBLK // NBUF, _group, 0)
        return carry

    lax.fori_loop(0, nch, _chunk, 0)
    plsc.subcore_barrier()
    pltpu.sync_copy(acc_sh.at[pl.ds(base, RPT)],
                    y_hbm.at[c, pl.ds(base, RPT)])


def _merge_chunk(yp_hbm, y_hbm, p0_v, p1_v, pt_v, nb, rows):
    pltpu.sync_copy(yp_hbm.at[0, pl.ds(nb, rows)], p0_v.at[pl.ds(0, rows)])
    pltpu.sync_copy(yp_hbm.at[1, pl.ds(nb, rows)], p1_v.at[pl.ds(0, rows)])
    lo = lax.iota(jnp.int32, 16)
    hi = lo + 16

    def _row(r, carry):
        v0 = p0_v[r, pl.ds(0, 16)] + p1_v[r, pl.ds(0, 16)]
        v1 = p0_v[r, pl.ds(16, 16)] + p1_v[r, pl.ds(16, 16)]
        rr = jnp.full((16,), 0, jnp.int32) + r
        plsc.store_scatter(pt_v, [lo, rr], v0)
        plsc.store_scatter(pt_v, [hi, rr], v1)
        return carry

    lax.fori_loop(0, rows, _row, 0)
    for b in range(B):
        pltpu.sync_copy(pt_v.at[pl.ds(b * OUT, OUT), pl.ds(0, rows)],
                        y_hbm.at[b, pl.ds(0, OUT), pl.ds(nb, rows)])


@functools.partial(
    pl.kernel,
    out_type=jax.ShapeDtypeStruct((B, OUT, N), jnp.float32),
    mesh=_mesh,
    scratch_types=[
        pltpu.VMEM((MC, D), jnp.float32),
        pltpu.VMEM((MC, D), jnp.float32),
        pltpu.VMEM((D, MC), jnp.float32),
    ],
    compiler_params=pltpu.CompilerParams(use_tc_tiling_on_sc=False,
                                         needs_layout_passes=False),
)
def _sc_merge(yp_hbm, y_hbm, p0_v, p1_v, pt_v):
    c = lax.axis_index("c")
    s = lax.axis_index("s")
    wid = c * NS + s

    @pl.when(wid < OCTX)
    def _():
        nb = wid * (OCT + 1) * 8
        _merge_chunk(yp_hbm, y_hbm, p0_v, p1_v, pt_v, nb, MC)
        _merge_chunk(yp_hbm, y_hbm, p0_v, p1_v, pt_v, nb + MC, MC)

    @pl.when(wid >= OCTX)
    def _():
        nb = (wid * OCT + OCTX) * 8
        _merge_chunk(yp_hbm, y_hbm, p0_v, p1_v, pt_v, nb, MC)
        _merge_chunk(yp_hbm, y_hbm, p0_v, p1_v, pt_v, nb + MC, MC - 8)


def kernel(input, edge_index, edge_types, weight_vector, bias_vector):
    x = input.astype(jnp.float32)
    W = weight_vector.reshape(T, OUT, IN)
    bias = bias_vector.reshape(T, OUT)

    Wt = jnp.transpose(W, (2, 0, 1))
    eye = jnp.eye(B, dtype=jnp.float32)
    wfull = eye[:, None, None, :, None] * Wt[None, :, :, None, :]
    wmat = wfull.reshape(B * IN, T * B * OUT)
    brow = jnp.broadcast_to(bias[:, None, :], (T, B, OUT)).reshape(1, T * B * OUT)

    x2 = jnp.pad(jnp.transpose(x, (0, 2, 1)).reshape(B * IN, N),
                 ((0, 0), (0, NP - N)))
    xw = _xw_call(x2, wmat, brow).reshape(NP * T, D)

    ei3 = jnp.transpose(edge_index.astype(jnp.int32).reshape(2, NBLKG, KB),
                        (1, 0, 2))
    tp = edge_types.astype(jnp.int32).reshape(NBLKG, KB)

    ypart = _sc_accum(xw, ei3, tp)
    return jnp.transpose(_sc_merge(ypart), (0, 2, 1))
